# baseline (speedup 1.0000x reference)
import functools

import jax
import jax.numpy as jnp
from jax import lax
from jax.experimental import pallas as pl
from jax.experimental.pallas import tpu as pltpu
from jax.experimental.pallas import tpu_sc as plsc

N = 10000
E = 160000
K = 5
C = 128
NW = 32
EP = 163840
NCH = EP // C
CPW = NCH // NW
NP = 10240
ROWS = NP // 16

ROW_BLK = 400


def _prep_body(src_ref, a0_ref, a1_ref, a2_ref, b_ref, idn_ref, iw_ref):
    src = src_ref[:, 0, :]
    fr = []
    fi = []
    for a_ref in (a0_ref, a1_ref, a2_ref):
        v = a_ref[:, 0, :] * (K - 1.0)
        fl = jnp.floor(v)
        fr.append(v - fl)
        fi.append(fl.astype(jnp.int32))
    for s in range(8):
        b = None
        wi = None
        stride = 1
        for d in range(3):
            bit = (s >> d) & 1
            bd = fr[d] if bit else 1.0 - fr[d]
            wd = jnp.clip(fi[d] + bit, 0, K - 1) * stride
            b = bd if b is None else b * bd
            wi = wd if wi is None else wi + wd
            stride *= K
        b_ref[:, s, :] = b
        iw_ref[:, s, :] = wi
        idn_ref[:, s, :] = jnp.minimum(src, N - 1) * 125 + wi


def _prep(src3, a0, a1, a2):
    blk = 10
    return pl.pallas_call(
        _prep_body,
        grid=(NCH // blk,),
        in_specs=[pl.BlockSpec((blk, 1, C), lambda i: (i, 0, 0))] * 4,
        out_specs=[pl.BlockSpec((blk, 8, C), lambda i: (i, 0, 0))] * 3,
        out_shape=[
            jax.ShapeDtypeStruct((NCH, 8, C), jnp.float32),
            jax.ShapeDtypeStruct((NCH, 8, C), jnp.int32),
            jax.ShapeDtypeStruct((NCH, 8, C), jnp.int32),
        ],
    )(src3, a0, a1, a2)


_MESH = plsc.VectorSubcoreMesh(core_axis_name="c", subcore_axis_name="s")


def _worker_id():
    return lax.axis_index("s") * 2 + lax.axis_index("c")


@functools.partial(
    pl.kernel,
    out_type=jax.ShapeDtypeStruct((2, NP, 16), jnp.float32),
    mesh=_MESH,
    compiler_params=pltpu.CompilerParams(use_tc_tiling_on_sc=False),
    scratch_types=[
        pltpu.VMEM((C,), jnp.int32),
        pltpu.VMEM((C, 16), jnp.float32),
        pltpu.VMEM_SHARED((NP, 16), jnp.float32),
    ],
)
def _deg_kernel(srcf_hbm, ones_hbm, zer_hbm, degp_hbm, src_v, ones_v, dacc):
    cid = lax.axis_index("c")
    sid = lax.axis_index("s")
    w = _worker_id()
    pltpu.sync_copy(ones_hbm, ones_v)
    pltpu.sync_copy(zer_hbm.at[pl.ds(sid * ROWS, ROWS)],
                    dacc.at[pl.ds(sid * ROWS, ROWS)])
    plsc.subcore_barrier()

    def chunk(i, carry):
        ch = w + i * NW
        pltpu.sync_copy(srcf_hbm.at[pl.ds(ch * C, C)], src_v)
        pltpu.sync_copy(ones_v, dacc.at[src_v], add=True)
        return carry

    lax.fori_loop(0, CPW, chunk, 0)
    plsc.subcore_barrier()
    pltpu.sync_copy(dacc.at[pl.ds(sid * ROWS, ROWS)],
                    degp_hbm.at[cid, pl.ds(sid * ROWS, ROWS)])


def _make_sc_layer(D):

    @functools.partial(
        pl.kernel,
        out_type=jax.ShapeDtypeStruct((2, NP, D), jnp.float32),
        mesh=_MESH,
        compiler_params=pltpu.CompilerParams(use_tc_tiling_on_sc=False),
        scratch_types=[
            pltpu.VMEM((8 * C,), jnp.int32),
            pltpu.VMEM((8 * C,), jnp.float32),
            pltpu.VMEM((C,), jnp.int32),
            [pltpu.VMEM((C, D), jnp.float32) for _ in range(8)],
            pltpu.VMEM((C, D), jnp.float32),
            pltpu.VMEM_SHARED((NP, D), jnp.float32),
            pltpu.SemaphoreType.DMA,
        ],
    )
    def sc_layer(table_hbm, idx_hbm, b_hbm, dst_hbm, zer_hbm, agg_hbm,
                 idx_v, b_v, dst_v, gb, msg_v, acc, sem):
        cid = lax.axis_index("c")
        sid = lax.axis_index("s")
        w = _worker_id()
        pltpu.sync_copy(zer_hbm.at[pl.ds(sid * ROWS, ROWS)],
                        acc.at[pl.ds(sid * ROWS, ROWS)])
        plsc.subcore_barrier()

        def chunk(i, carry):
            ch = w + i * NW
            pltpu.sync_copy(idx_hbm.at[pl.ds(ch * (8 * C), 8 * C)], idx_v)
            pltpu.sync_copy(b_hbm.at[pl.ds(ch * (8 * C), 8 * C)], b_v)
            pltpu.sync_copy(dst_hbm.at[pl.ds(ch * C, C)], dst_v)
            cps = [
                pltpu.async_copy(
                    table_hbm.at[idx_v.at[pl.ds(s * C, C)]], gb[s], sem)
                for s in range(8)
            ]
            for cp in cps:
                cp.wait()

            def egroup(eg, ecarry):
                bvs = [b_v[pl.ds(s * C + eg * 16, 16)] for s in range(8)]
                for j in range(16):
                    e = eg * 16 + j
                    bbs = [lax.broadcast(bvs[s][j], (16,)) for s in range(8)]
                    for g in range(D // 16):
                        sl = pl.ds(g * 16, 16)
                        accv = bbs[0] * gb[0][e, sl]
                        for s in range(1, 8):
                            accv = accv + bbs[s] * gb[s][e, sl]
                        msg_v[e, sl] = accv
                return ecarry

            lax.fori_loop(0, C // 16, egroup, 0)
            pltpu.sync_copy(msg_v, acc.at[dst_v], add=True)
            return carry

        lax.fori_loop(0, CPW, chunk, 0)
        plsc.subcore_barrier()
        pltpu.sync_copy(acc.at[pl.ds(sid * ROWS, ROWS)],
                        agg_hbm.at[cid, pl.ds(sid * ROWS, ROWS)])

    return sc_layer


_sc_layer64 = _make_sc_layer(64)


@functools.partial(
    pl.kernel,
    out_type=jax.ShapeDtypeStruct((2, NP, 32), jnp.float32),
    mesh=_MESH,
    compiler_params=pltpu.CompilerParams(use_tc_tiling_on_sc=False),
    scratch_types=[
        pltpu.VMEM((125, 32), jnp.float32),
        pltpu.VMEM((8 * C,), jnp.int32),
        pltpu.VMEM((8 * C,), jnp.float32),
        pltpu.VMEM((C,), jnp.int32),
        pltpu.VMEM((C, 32), jnp.float32),
        pltpu.VMEM_SHARED((NP, 32), jnp.float32),
    ],
)
def _sc_layer1(table_hbm, idx_hbm, b_hbm, dst_hbm, zer_hbm, agg_hbm,
               tb, idx_v, b_v, dst_v, msg_v, acc, sem=None):
    cid = lax.axis_index("c")
    sid = lax.axis_index("s")
    w = _worker_id()
    pltpu.sync_copy(table_hbm, tb)
    pltpu.sync_copy(zer_hbm.at[pl.ds(sid * ROWS, ROWS)],
                    acc.at[pl.ds(sid * ROWS, ROWS)])
    plsc.subcore_barrier()

    def chunk(i, carry):
        ch = w + i * NW
        pltpu.sync_copy(idx_hbm.at[pl.ds(ch * (8 * C), 8 * C)], idx_v)
        pltpu.sync_copy(b_hbm.at[pl.ds(ch * (8 * C), 8 * C)], b_v)
        pltpu.sync_copy(dst_hbm.at[pl.ds(ch * C, C)], dst_v)

        def egroup(eg, ecarry):
            bvs = [b_v[pl.ds(s * C + eg * 16, 16)] for s in range(8)]
            wvs = [idx_v[pl.ds(s * C + eg * 16, 16)] for s in range(8)]
            for j in range(16):
                e = eg * 16 + j
                bbs = [lax.broadcast(bvs[s][j], (16,)) for s in range(8)]
                wis = [wvs[s][j] for s in range(8)]
                for g in range(2):
                    sl = pl.ds(g * 16, 16)
                    accv = bbs[0] * tb[wis[0], sl]
                    for s in range(1, 8):
                        accv = accv + bbs[s] * tb[wis[s], sl]
                    msg_v[e, sl] = accv
            return ecarry

        lax.fori_loop(0, C // 16, egroup, 0)
        pltpu.sync_copy(msg_v, acc.at[dst_v], add=True)
        return carry

    lax.fori_loop(0, CPW, chunk, 0)
    plsc.subcore_barrier()
    pltpu.sync_copy(acc.at[pl.ds(sid * ROWS, ROWS)],
                    agg_hbm.at[cid, pl.ds(sid * ROWS, ROWS)])


def _mm_body(h_ref, w_ref, out_ref):
    out_ref[...] = jnp.dot(h_ref[...], w_ref[...],
                           preferred_element_type=jnp.float32)


def _mm(h, Wr):
    din = h.shape[1]
    cols = Wr.shape[1]
    return pl.pallas_call(
        _mm_body,
        grid=(N // ROW_BLK,),
        in_specs=[
            pl.BlockSpec((ROW_BLK, din), lambda i: (i, 0)),
            pl.BlockSpec((din, cols), lambda i: (0, 0)),
        ],
        out_specs=pl.BlockSpec((ROW_BLK, cols), lambda i: (i, 0)),
        out_shape=jax.ShapeDtypeStruct((N, cols), jnp.float32),
    )(h, Wr)


def _act_body(h_ref, agg_ref, root_ref, bias_ref, degp_ref, out_ref):
    deg = 1.0 + degp_ref[0, :, 0:1] + degp_ref[1, :, 0:1]
    a = agg_ref[0] + agg_ref[1] + bias_ref[...]
    a = a + jnp.dot(h_ref[...], root_ref[...],
                    preferred_element_type=jnp.float32)
    a = a / deg
    out_ref[...] = jnp.where(a > 0, a, jnp.exp(a) - 1.0)


def _act(h, agg, root, bias, degp):
    din = h.shape[1]
    d = agg.shape[2]
    return pl.pallas_call(
        _act_body,
        in_specs=[
            pl.BlockSpec((N, din), lambda: (0, 0)),
            pl.BlockSpec((2, N, d), lambda: (0, 0, 0)),
            pl.BlockSpec((din, d), lambda: (0, 0)),
            pl.BlockSpec((d,), lambda: (0,)),
            pl.BlockSpec((2, N, 16), lambda: (0, 0, 0)),
        ],
        out_specs=pl.BlockSpec((N, d), lambda: (0, 0)),
        out_shape=jax.ShapeDtypeStruct((N, d), jnp.float32),
        grid=(),
    )(h, agg, root, bias, degp)


def _head_body(h_ref, fc1w_ref, fc1b_ref, fc2w_ref, fc2b_ref, out_ref):
    h = h_ref[...]
    a = jnp.dot(h, fc1w_ref[...], preferred_element_type=jnp.float32)
    a = a + fc1b_ref[...]
    a = jnp.where(a > 0, a, jnp.exp(a) - 1.0)
    logits = jnp.dot(a.astype(jnp.bfloat16), fc2w_ref[...],
                      preferred_element_type=jnp.float32)
    logits = logits + fc2b_ref[...]
    m = jnp.max(logits, axis=1, keepdims=True)
    lse = jnp.log(jnp.sum(jnp.exp(logits - m), axis=1, keepdims=True))
    out_ref[...] = logits - m - lse


def _head(h, fc1_w, fc1_b, fc2_w, fc2_b):
    return pl.pallas_call(
        _head_body,
        grid=(N // ROW_BLK,),
        in_specs=[
            pl.BlockSpec((ROW_BLK, 64), lambda i: (i, 0)),
            pl.BlockSpec((64, 256), lambda i: (0, 0)),
            pl.BlockSpec((256,), lambda i: (0,)),
            pl.BlockSpec((256, N), lambda i: (0, 0)),
            pl.BlockSpec((N,), lambda i: (0,)),
        ],
        out_specs=pl.BlockSpec((ROW_BLK, N), lambda i: (i, 0)),
        out_shape=jax.ShapeDtypeStruct((N, N), jnp.float32),
    )(h, fc1_w, fc1_b, fc2_w.astype(jnp.bfloat16), fc2_b)


def kernel(x, edge_index, edge_attr, W1, root1, b1, W2, root2, b2, W3, root3,
           b3, W4, root4, b4, W5, root5, b5, W6, root6, b6, fc1_w, fc1_b,
           fc2_w, fc2_b):
    pad_i = N + (jnp.arange(EP - E, dtype=jnp.int32) % (NP - N))
    src = jnp.concatenate([edge_index[0], pad_i])
    dst = jnp.concatenate([edge_index[1], pad_i])
    ea = jnp.concatenate([edge_attr, jnp.zeros((EP - E, 3), jnp.float32)])
    src3 = src.reshape(NCH, 1, C)
    a0 = ea[:, 0].reshape(NCH, 1, C)
    a1 = ea[:, 1].reshape(NCH, 1, C)
    a2 = ea[:, 2].reshape(NCH, 1, C)

    b8, idn, iw = _prep(src3, a0, a1, a2)
    b8f = b8.reshape(8 * EP)
    idnf = idn.reshape(8 * EP)
    iwf = iw.reshape(8 * EP)

    zer16 = jnp.zeros((NP, 16), jnp.float32)
    zer32 = jnp.zeros((NP, 32), jnp.float32)
    zer64 = jnp.zeros((NP, 64), jnp.float32)
    ones16 = jnp.ones((C, 16), jnp.float32)

    degp = _deg_kernel(src, ones16, zer16)[:, :N, :]

    agg = _sc_layer1(W1[:, 0, :], iwf, b8f, dst, zer32)[:, :N, :]
    h = _act(x, agg, root1, b1, degp)

    for (W, root, b) in ((W2, root2, b2), (W3, root3, b3), (W4, root4, b4),
                         (W5, root5, b5), (W6, root6, b6)):
        din = W.shape[1]
        Wr = W.transpose(1, 0, 2).reshape(din, 125 * 64)
        T = _mm(h, Wr).reshape(N * 125, 64)
        agg = _sc_layer64(T, idnf, b8f, dst, zer64)[:, :N, :]
        h = _act(h, agg, root, b, degp)

    return _head(h, fc1_w, fc1_b, fc2_w, fc2_b)


# speedup vs baseline: 1.1621x; 1.1621x over previous
import functools

import jax
import jax.numpy as jnp
from jax import lax
from jax.experimental import pallas as pl
from jax.experimental.pallas import tpu as pltpu
from jax.experimental.pallas import tpu_sc as plsc

N = 10000
E = 160000
K = 5
C = 128
NW = 32
NCH = E // C
NP = 10240
ROWS = NP // 16

ROW_BLK = 400


def _prep_body(src_ref, a0_ref, a1_ref, a2_ref, b_ref, idn_ref, iw_ref):
    src = src_ref[:, 0, :]
    fr = []
    fi = []
    for a_ref in (a0_ref, a1_ref, a2_ref):
        v = a_ref[:, 0, :] * (K - 1.0)
        fl = jnp.floor(v)
        fr.append(v - fl)
        fi.append(fl.astype(jnp.int32))
    for s in range(8):
        b = None
        wi = None
        stride = 1
        for d in range(3):
            bit = (s >> d) & 1
            bd = fr[d] if bit else 1.0 - fr[d]
            wd = jnp.clip(fi[d] + bit, 0, K - 1) * stride
            b = bd if b is None else b * bd
            wi = wd if wi is None else wi + wd
            stride *= K
        b_ref[:, s, :] = b
        iw_ref[:, s, :] = wi
        idn_ref[:, s, :] = jnp.minimum(src, N - 1) * 125 + wi


def _prep(src3, a0, a1, a2):
    blk = 10
    return pl.pallas_call(
        _prep_body,
        grid=(NCH // blk,),
        in_specs=[pl.BlockSpec((blk, 1, C), lambda i: (i, 0, 0))] * 4,
        out_specs=[pl.BlockSpec((blk, 8, C), lambda i: (i, 0, 0))] * 3,
        out_shape=[
            jax.ShapeDtypeStruct((NCH, 8, C), jnp.float32),
            jax.ShapeDtypeStruct((NCH, 8, C), jnp.int32),
            jax.ShapeDtypeStruct((NCH, 8, C), jnp.int32),
        ],
    )(src3, a0, a1, a2)


_MESH = plsc.VectorSubcoreMesh(core_axis_name="c", subcore_axis_name="s")


def _worker_id():
    return lax.axis_index("s") * 2 + lax.axis_index("c")


def _nchunks(w):
    return jnp.where(w < NCH - (NCH // NW) * NW, NCH // NW + 1, NCH // NW)


@functools.partial(
    pl.kernel,
    out_type=jax.ShapeDtypeStruct((2, NP, 16), jnp.float32),
    mesh=_MESH,
    compiler_params=pltpu.CompilerParams(use_tc_tiling_on_sc=False),
    scratch_types=[
        pltpu.VMEM((C,), jnp.int32),
        pltpu.VMEM((C, 16), jnp.float32),
        pltpu.VMEM_SHARED((NP, 16), jnp.float32),
    ],
)
def _deg_kernel(srcf_hbm, ones_hbm, zer_hbm, degp_hbm, src_v, ones_v, dacc):
    cid = lax.axis_index("c")
    sid = lax.axis_index("s")
    w = _worker_id()
    pltpu.sync_copy(ones_hbm, ones_v)
    pltpu.sync_copy(zer_hbm.at[pl.ds(sid * ROWS, ROWS)],
                    dacc.at[pl.ds(sid * ROWS, ROWS)])
    plsc.subcore_barrier()

    def chunk(i, carry):
        ch = w + i * NW
        pltpu.sync_copy(srcf_hbm.at[pl.ds(ch * C, C)], src_v)
        pltpu.sync_copy(ones_v, dacc.at[src_v], add=True)
        return carry

    lax.fori_loop(0, _nchunks(w), chunk, 0)
    plsc.subcore_barrier()
    pltpu.sync_copy(dacc.at[pl.ds(sid * ROWS, ROWS)],
                    degp_hbm.at[cid, pl.ds(sid * ROWS, ROWS)])


def _make_sc_layer(D):

    @functools.partial(
        pl.kernel,
        out_type=jax.ShapeDtypeStruct((2, NP, D), jnp.float32),
        mesh=_MESH,
        compiler_params=pltpu.CompilerParams(use_tc_tiling_on_sc=False),
        scratch_types=[
            pltpu.VMEM((8 * C,), jnp.int32),
            pltpu.VMEM((8 * C,), jnp.float32),
            pltpu.VMEM((C,), jnp.int32),
            [pltpu.VMEM((C, D), jnp.float32) for _ in range(8)],
            pltpu.VMEM((C, D), jnp.float32),
            pltpu.VMEM_SHARED((NP, D), jnp.float32),
            pltpu.SemaphoreType.DMA,
        ],
    )
    def sc_layer(table_hbm, idx_hbm, b_hbm, dst_hbm, zer_hbm, agg_hbm,
                 idx_v, b_v, dst_v, gb, msg_v, acc, sem):
        cid = lax.axis_index("c")
        sid = lax.axis_index("s")
        w = _worker_id()
        pltpu.sync_copy(zer_hbm.at[pl.ds(sid * ROWS, ROWS)],
                        acc.at[pl.ds(sid * ROWS, ROWS)])
        plsc.subcore_barrier()

        def chunk(i, carry):
            ch = w + i * NW
            pltpu.sync_copy(idx_hbm.at[pl.ds(ch * (8 * C), 8 * C)], idx_v)
            pltpu.sync_copy(b_hbm.at[pl.ds(ch * (8 * C), 8 * C)], b_v)
            pltpu.sync_copy(dst_hbm.at[pl.ds(ch * C, C)], dst_v)
            cps = [
                pltpu.async_copy(
                    table_hbm.at[idx_v.at[pl.ds(s * C, C)]], gb[s], sem)
                for s in range(8)
            ]
            for cp in cps:
                cp.wait()

            def egroup(eg, ecarry):
                bvs = [b_v[pl.ds(s * C + eg * 16, 16)] for s in range(8)]
                for j in range(16):
                    e = eg * 16 + j
                    bbs = [lax.broadcast(bvs[s][j], (16,)) for s in range(8)]
                    for g in range(D // 16):
                        sl = pl.ds(g * 16, 16)
                        accv = bbs[0] * gb[0][e, sl]
                        for s in range(1, 8):
                            accv = accv + bbs[s] * gb[s][e, sl]
                        msg_v[e, sl] = accv
                return ecarry

            lax.fori_loop(0, C // 16, egroup, 0)
            pltpu.sync_copy(msg_v, acc.at[dst_v], add=True)
            return carry

        lax.fori_loop(0, _nchunks(w), chunk, 0)
        plsc.subcore_barrier()
        pltpu.sync_copy(acc.at[pl.ds(sid * ROWS, ROWS)],
                        agg_hbm.at[cid, pl.ds(sid * ROWS, ROWS)])

    return sc_layer


_sc_layer64 = _make_sc_layer(64)


@functools.partial(
    pl.kernel,
    out_type=jax.ShapeDtypeStruct((2, NP, 32), jnp.float32),
    mesh=_MESH,
    compiler_params=pltpu.CompilerParams(use_tc_tiling_on_sc=False),
    scratch_types=[
        pltpu.VMEM((125, 32), jnp.float32),
        pltpu.VMEM((8 * C,), jnp.int32),
        pltpu.VMEM((8 * C,), jnp.float32),
        pltpu.VMEM((C,), jnp.int32),
        pltpu.VMEM((C, 32), jnp.float32),
        pltpu.VMEM_SHARED((NP, 32), jnp.float32),
    ],
)
def _sc_layer1(table_hbm, idx_hbm, b_hbm, dst_hbm, zer_hbm, agg_hbm,
               tb, idx_v, b_v, dst_v, msg_v, acc, sem=None):
    cid = lax.axis_index("c")
    sid = lax.axis_index("s")
    w = _worker_id()
    pltpu.sync_copy(table_hbm, tb)
    pltpu.sync_copy(zer_hbm.at[pl.ds(sid * ROWS, ROWS)],
                    acc.at[pl.ds(sid * ROWS, ROWS)])
    plsc.subcore_barrier()

    def chunk(i, carry):
        ch = w + i * NW
        pltpu.sync_copy(idx_hbm.at[pl.ds(ch * (8 * C), 8 * C)], idx_v)
        pltpu.sync_copy(b_hbm.at[pl.ds(ch * (8 * C), 8 * C)], b_v)
        pltpu.sync_copy(dst_hbm.at[pl.ds(ch * C, C)], dst_v)

        def egroup(eg, ecarry):
            bvs = [b_v[pl.ds(s * C + eg * 16, 16)] for s in range(8)]
            wvs = [idx_v[pl.ds(s * C + eg * 16, 16)] for s in range(8)]
            for j in range(16):
                e = eg * 16 + j
                bbs = [lax.broadcast(bvs[s][j], (16,)) for s in range(8)]
                wis = [wvs[s][j] for s in range(8)]
                for g in range(2):
                    sl = pl.ds(g * 16, 16)
                    accv = bbs[0] * tb[wis[0], sl]
                    for s in range(1, 8):
                        accv = accv + bbs[s] * tb[wis[s], sl]
                    msg_v[e, sl] = accv
            return ecarry

        lax.fori_loop(0, C // 16, egroup, 0)
        pltpu.sync_copy(msg_v, acc.at[dst_v], add=True)
        return carry

    lax.fori_loop(0, _nchunks(w), chunk, 0)
    plsc.subcore_barrier()
    pltpu.sync_copy(acc.at[pl.ds(sid * ROWS, ROWS)],
                    agg_hbm.at[cid, pl.ds(sid * ROWS, ROWS)])


def _mm_body(h_ref, w_ref, out_ref):
    out_ref[...] = jnp.dot(h_ref[...], w_ref[...],
                           preferred_element_type=jnp.float32)


def _mm(h, Wr):
    din = h.shape[1]
    cols = Wr.shape[1]
    return pl.pallas_call(
        _mm_body,
        grid=(N // ROW_BLK,),
        in_specs=[
            pl.BlockSpec((ROW_BLK, din), lambda i: (i, 0)),
            pl.BlockSpec((din, cols), lambda i: (0, 0)),
        ],
        out_specs=pl.BlockSpec((ROW_BLK, cols), lambda i: (i, 0)),
        out_shape=jax.ShapeDtypeStruct((N, cols), jnp.float32),
    )(h, Wr)


def _act_body(h_ref, agg_ref, root_ref, bias_ref, degp_ref, out_ref):
    deg = 1.0 + degp_ref[0, :, 0:1] + degp_ref[1, :, 0:1]
    a = agg_ref[0] + agg_ref[1] + bias_ref[...]
    a = a + jnp.dot(h_ref[...], root_ref[...],
                    preferred_element_type=jnp.float32)
    a = a / deg
    out_ref[...] = jnp.where(a > 0, a, jnp.exp(a) - 1.0)


def _act(h, agg, root, bias, degp):
    din = h.shape[1]
    d = agg.shape[2]
    return pl.pallas_call(
        _act_body,
        in_specs=[
            pl.BlockSpec((N, din), lambda: (0, 0)),
            pl.BlockSpec((2, N, d), lambda: (0, 0, 0)),
            pl.BlockSpec((din, d), lambda: (0, 0)),
            pl.BlockSpec((d,), lambda: (0,)),
            pl.BlockSpec((2, N, 16), lambda: (0, 0, 0)),
        ],
        out_specs=pl.BlockSpec((N, d), lambda: (0, 0)),
        out_shape=jax.ShapeDtypeStruct((N, d), jnp.float32),
        grid=(),
    )(h, agg, root, bias, degp)


def _head_body(h_ref, fc1w_ref, fc1b_ref, fc2w_ref, fc2b_ref, out_ref):
    h = h_ref[...]
    a = jnp.dot(h, fc1w_ref[...], preferred_element_type=jnp.float32)
    a = a + fc1b_ref[...]
    a = jnp.where(a > 0, a, jnp.exp(a) - 1.0)
    logits = jnp.dot(a.astype(jnp.bfloat16), fc2w_ref[...],
                      preferred_element_type=jnp.float32)
    logits = logits + fc2b_ref[...]
    m = jnp.max(logits, axis=1, keepdims=True)
    lse = jnp.log(jnp.sum(jnp.exp(logits - m), axis=1, keepdims=True))
    out_ref[...] = logits - m - lse


def _head(h, fc1_w, fc1_b, fc2_w, fc2_b):
    return pl.pallas_call(
        _head_body,
        grid=(N // ROW_BLK,),
        in_specs=[
            pl.BlockSpec((ROW_BLK, 64), lambda i: (i, 0)),
            pl.BlockSpec((64, 256), lambda i: (0, 0)),
            pl.BlockSpec((256,), lambda i: (0,)),
            pl.BlockSpec((256, N), lambda i: (0, 0)),
            pl.BlockSpec((N,), lambda i: (0,)),
        ],
        out_specs=pl.BlockSpec((ROW_BLK, N), lambda i: (i, 0)),
        out_shape=jax.ShapeDtypeStruct((N, N), jnp.float32),
    )(h, fc1_w, fc1_b, fc2_w.astype(jnp.bfloat16), fc2_b)


def kernel(x, edge_index, edge_attr, W1, root1, b1, W2, root2, b2, W3, root3,
           b3, W4, root4, b4, W5, root5, b5, W6, root6, b6, fc1_w, fc1_b,
           fc2_w, fc2_b):
    src = edge_index[0]
    dst = edge_index[1]
    src3 = src.reshape(NCH, 1, C)
    a0 = edge_attr[:, 0].reshape(NCH, 1, C)
    a1 = edge_attr[:, 1].reshape(NCH, 1, C)
    a2 = edge_attr[:, 2].reshape(NCH, 1, C)

    b8, idn, iw = _prep(src3, a0, a1, a2)
    b8f = b8.reshape(8 * E)
    idnf = idn.reshape(8 * E)
    iwf = iw.reshape(8 * E)

    zer16 = jnp.zeros((NP, 16), jnp.float32)
    zer32 = jnp.zeros((NP, 32), jnp.float32)
    zer64 = jnp.zeros((NP, 64), jnp.float32)
    ones16 = jnp.ones((C, 16), jnp.float32)

    degp = _deg_kernel(src, ones16, zer16)[:, :N, :]

    agg = _sc_layer1(W1[:, 0, :], iwf, b8f, dst, zer32)[:, :N, :]
    h = _act(x, agg, root1, b1, degp)

    for (W, root, b) in ((W2, root2, b2), (W3, root3, b3), (W4, root4, b4),
                         (W5, root5, b5), (W6, root6, b6)):
        din = W.shape[1]
        Wr = W.transpose(1, 0, 2).reshape(din, 125 * 64)
        T = _mm(h, Wr).reshape(N * 125, 64)
        agg = _sc_layer64(T, idnf, b8f, dst, zer64)[:, :N, :]
        h = _act(h, agg, root, b, degp)

    return _head(h, fc1_w, fc1_b, fc2_w, fc2_b)


# speedup vs baseline: 1.2108x; 1.0418x over previous
import functools

import jax
import jax.numpy as jnp
from jax import lax
from jax.experimental import pallas as pl
from jax.experimental.pallas import tpu as pltpu
from jax.experimental.pallas import tpu_sc as plsc

N = 10000
E = 160000
K = 5
C = 128
NW = 32
NCH = E // C
NP = 10240
ROWS = NP // 16

ROW_BLK = 400


def _prep_body(src_ref, a0_ref, a1_ref, a2_ref, b_ref, idn_ref, iw_ref):
    src = src_ref[:, 0, :]
    fr = []
    fi = []
    for a_ref in (a0_ref, a1_ref, a2_ref):
        v = a_ref[:, 0, :] * (K - 1.0)
        fl = jnp.floor(v)
        fr.append(v - fl)
        fi.append(fl.astype(jnp.int32))
    for s in range(8):
        b = None
        wi = None
        stride = 1
        for d in range(3):
            bit = (s >> d) & 1
            bd = fr[d] if bit else 1.0 - fr[d]
            wd = jnp.clip(fi[d] + bit, 0, K - 1) * stride
            b = bd if b is None else b * bd
            wi = wd if wi is None else wi + wd
            stride *= K
        b_ref[:, s, :] = b
        iw_ref[:, s, :] = wi
        idn_ref[:, s, :] = jnp.minimum(src, N - 1) * 125 + wi


def _prep(src3, a0, a1, a2):
    blk = 10
    return pl.pallas_call(
        _prep_body,
        grid=(NCH // blk,),
        in_specs=[pl.BlockSpec((blk, 1, C), lambda i: (i, 0, 0))] * 4,
        out_specs=[pl.BlockSpec((blk, 8, C), lambda i: (i, 0, 0))] * 3,
        out_shape=[
            jax.ShapeDtypeStruct((NCH, 8, C), jnp.float32),
            jax.ShapeDtypeStruct((NCH, 8, C), jnp.int32),
            jax.ShapeDtypeStruct((NCH, 8, C), jnp.int32),
        ],
    )(src3, a0, a1, a2)


_MESH = plsc.VectorSubcoreMesh(core_axis_name="c", subcore_axis_name="s")


def _worker_id():
    return lax.axis_index("s") * 2 + lax.axis_index("c")


def _nchunks(w):
    return jnp.where(w < NCH - (NCH // NW) * NW, NCH // NW + 1, NCH // NW)


@functools.partial(
    pl.kernel,
    out_type=jax.ShapeDtypeStruct((2, NP, 16), jnp.float32),
    mesh=_MESH,
    compiler_params=pltpu.CompilerParams(use_tc_tiling_on_sc=False),
    scratch_types=[
        pltpu.VMEM((C,), jnp.int32),
        pltpu.VMEM((C, 16), jnp.float32),
        pltpu.VMEM_SHARED((NP, 16), jnp.float32),
    ],
)
def _deg_kernel(srcf_hbm, ones_hbm, zer_hbm, degp_hbm, src_v, ones_v, dacc):
    cid = lax.axis_index("c")
    sid = lax.axis_index("s")
    w = _worker_id()
    pltpu.sync_copy(ones_hbm, ones_v)
    pltpu.sync_copy(zer_hbm.at[pl.ds(sid * ROWS, ROWS)],
                    dacc.at[pl.ds(sid * ROWS, ROWS)])
    plsc.subcore_barrier()

    def chunk(i, carry):
        ch = w + i * NW
        pltpu.sync_copy(srcf_hbm.at[pl.ds(ch * C, C)], src_v)
        pltpu.sync_copy(ones_v, dacc.at[src_v], add=True)
        return carry

    lax.fori_loop(0, _nchunks(w), chunk, 0)
    plsc.subcore_barrier()
    pltpu.sync_copy(dacc.at[pl.ds(sid * ROWS, ROWS)],
                    degp_hbm.at[cid, pl.ds(sid * ROWS, ROWS)])


def _make_sc_layer(D):

    @functools.partial(
        pl.kernel,
        out_type=jax.ShapeDtypeStruct((2, NP, D), jnp.float32),
        mesh=_MESH,
        compiler_params=pltpu.CompilerParams(use_tc_tiling_on_sc=False),
        scratch_types=[
            pltpu.VMEM((8 * C,), jnp.int32),
            pltpu.VMEM((8 * C,), jnp.float32),
            pltpu.VMEM((C,), jnp.int32),
            [pltpu.VMEM((C, D), jnp.float32) for _ in range(8)],
            pltpu.VMEM((C, D), jnp.float32),
            pltpu.VMEM_SHARED((NP, D), jnp.float32),
            pltpu.SemaphoreType.DMA,
            pltpu.SemaphoreType.DMA,
        ],
    )
    def sc_layer(table_hbm, idx_hbm, b_hbm, dst_hbm, zer_hbm, agg_hbm,
                 idx_v, b_v, dst_v, gb, msg_v, acc, sem, msem):
        cid = lax.axis_index("c")
        sid = lax.axis_index("s")
        w = _worker_id()
        pltpu.sync_copy(zer_hbm.at[pl.ds(sid * ROWS, ROWS)],
                        acc.at[pl.ds(sid * ROWS, ROWS)])
        plsc.subcore_barrier()

        def chunk(i, carry):
            ch = w + i * NW
            mps = [
                pltpu.async_copy(
                    idx_hbm.at[pl.ds(ch * (8 * C), 8 * C)], idx_v, msem),
                pltpu.async_copy(
                    b_hbm.at[pl.ds(ch * (8 * C), 8 * C)], b_v, msem),
                pltpu.async_copy(dst_hbm.at[pl.ds(ch * C, C)], dst_v, msem),
            ]
            for mp in mps:
                mp.wait()
            cps = [
                pltpu.async_copy(
                    table_hbm.at[idx_v.at[pl.ds(s * C, C)]], gb[s], sem)
                for s in range(8)
            ]
            for cp in cps:
                cp.wait()

            def egroup(eg, ecarry):
                bvs = [b_v[pl.ds(s * C + eg * 16, 16)] for s in range(8)]
                for j in range(16):
                    e = eg * 16 + j
                    bbs = [lax.broadcast(bvs[s][j], (16,)) for s in range(8)]
                    for g in range(D // 16):
                        sl = pl.ds(g * 16, 16)
                        accv = bbs[0] * gb[0][e, sl]
                        for s in range(1, 8):
                            accv = accv + bbs[s] * gb[s][e, sl]
                        msg_v[e, sl] = accv
                return ecarry

            lax.fori_loop(0, C // 16, egroup, 0)
            pltpu.sync_copy(msg_v, acc.at[dst_v], add=True)
            return carry

        lax.fori_loop(0, _nchunks(w), chunk, 0)
        plsc.subcore_barrier()
        pltpu.sync_copy(acc.at[pl.ds(sid * ROWS, ROWS)],
                        agg_hbm.at[cid, pl.ds(sid * ROWS, ROWS)])

    return sc_layer


_sc_layer64 = _make_sc_layer(64)


@functools.partial(
    pl.kernel,
    out_type=jax.ShapeDtypeStruct((2, NP, 32), jnp.float32),
    mesh=_MESH,
    compiler_params=pltpu.CompilerParams(use_tc_tiling_on_sc=False),
    scratch_types=[
        pltpu.VMEM((125, 32), jnp.float32),
        pltpu.VMEM((8 * C,), jnp.int32),
        pltpu.VMEM((8 * C,), jnp.float32),
        pltpu.VMEM((C,), jnp.int32),
        pltpu.VMEM((C, 32), jnp.float32),
        pltpu.VMEM_SHARED((NP, 32), jnp.float32),
    ],
)
def _sc_layer1(table_hbm, idx_hbm, b_hbm, dst_hbm, zer_hbm, agg_hbm,
               tb, idx_v, b_v, dst_v, msg_v, acc, sem=None):
    cid = lax.axis_index("c")
    sid = lax.axis_index("s")
    w = _worker_id()
    pltpu.sync_copy(table_hbm, tb)
    pltpu.sync_copy(zer_hbm.at[pl.ds(sid * ROWS, ROWS)],
                    acc.at[pl.ds(sid * ROWS, ROWS)])
    plsc.subcore_barrier()

    def chunk(i, carry):
        ch = w + i * NW
        pltpu.sync_copy(idx_hbm.at[pl.ds(ch * (8 * C), 8 * C)], idx_v)
        pltpu.sync_copy(b_hbm.at[pl.ds(ch * (8 * C), 8 * C)], b_v)
        pltpu.sync_copy(dst_hbm.at[pl.ds(ch * C, C)], dst_v)

        def egroup(eg, ecarry):
            bvs = [b_v[pl.ds(s * C + eg * 16, 16)] for s in range(8)]
            wvs = [idx_v[pl.ds(s * C + eg * 16, 16)] for s in range(8)]
            for j in range(16):
                e = eg * 16 + j
                bbs = [lax.broadcast(bvs[s][j], (16,)) for s in range(8)]
                wis = [wvs[s][j] for s in range(8)]
                for g in range(2):
                    sl = pl.ds(g * 16, 16)
                    accv = bbs[0] * tb[wis[0], sl]
                    for s in range(1, 8):
                        accv = accv + bbs[s] * tb[wis[s], sl]
                    msg_v[e, sl] = accv
            return ecarry

        lax.fori_loop(0, C // 16, egroup, 0)
        pltpu.sync_copy(msg_v, acc.at[dst_v], add=True)
        return carry

    lax.fori_loop(0, _nchunks(w), chunk, 0)
    plsc.subcore_barrier()
    pltpu.sync_copy(acc.at[pl.ds(sid * ROWS, ROWS)],
                    agg_hbm.at[cid, pl.ds(sid * ROWS, ROWS)])


def _mm_body(h_ref, w_ref, out_ref):
    out_ref[...] = jnp.dot(h_ref[...], w_ref[...],
                           preferred_element_type=jnp.float32)


def _mm(h, Wr):
    din = h.shape[1]
    cols = Wr.shape[1]
    return pl.pallas_call(
        _mm_body,
        grid=(N // ROW_BLK,),
        in_specs=[
            pl.BlockSpec((ROW_BLK, din), lambda i: (i, 0)),
            pl.BlockSpec((din, cols), lambda i: (0, 0)),
        ],
        out_specs=pl.BlockSpec((ROW_BLK, cols), lambda i: (i, 0)),
        out_shape=jax.ShapeDtypeStruct((N, cols), jnp.float32),
    )(h, Wr)


def _act_body(h_ref, agg_ref, root_ref, bias_ref, degp_ref, out_ref):
    deg = 1.0 + degp_ref[0, :, 0:1] + degp_ref[1, :, 0:1]
    a = agg_ref[0] + agg_ref[1] + bias_ref[...]
    a = a + jnp.dot(h_ref[...], root_ref[...],
                    preferred_element_type=jnp.float32)
    a = a / deg
    out_ref[...] = jnp.where(a > 0, a, jnp.exp(a) - 1.0)


def _act(h, agg, root, bias, degp):
    din = h.shape[1]
    d = agg.shape[2]
    return pl.pallas_call(
        _act_body,
        in_specs=[
            pl.BlockSpec((N, din), lambda: (0, 0)),
            pl.BlockSpec((2, N, d), lambda: (0, 0, 0)),
            pl.BlockSpec((din, d), lambda: (0, 0)),
            pl.BlockSpec((d,), lambda: (0,)),
            pl.BlockSpec((2, N, 16), lambda: (0, 0, 0)),
        ],
        out_specs=pl.BlockSpec((N, d), lambda: (0, 0)),
        out_shape=jax.ShapeDtypeStruct((N, d), jnp.float32),
        grid=(),
    )(h, agg, root, bias, degp)


def _head_body(h_ref, fc1w_ref, fc1b_ref, fc2w_ref, fc2b_ref, out_ref):
    h = h_ref[...]
    a = jnp.dot(h, fc1w_ref[...], preferred_element_type=jnp.float32)
    a = a + fc1b_ref[...]
    a = jnp.where(a > 0, a, jnp.exp(a) - 1.0)
    logits = jnp.dot(a.astype(jnp.bfloat16), fc2w_ref[...],
                      preferred_element_type=jnp.float32)
    logits = logits + fc2b_ref[...]
    m = jnp.max(logits, axis=1, keepdims=True)
    lse = jnp.log(jnp.sum(jnp.exp(logits - m), axis=1, keepdims=True))
    out_ref[...] = logits - m - lse


def _head(h, fc1_w, fc1_b, fc2_w, fc2_b):
    return pl.pallas_call(
        _head_body,
        grid=(N // ROW_BLK,),
        in_specs=[
            pl.BlockSpec((ROW_BLK, 64), lambda i: (i, 0)),
            pl.BlockSpec((64, 256), lambda i: (0, 0)),
            pl.BlockSpec((256,), lambda i: (0,)),
            pl.BlockSpec((256, N), lambda i: (0, 0)),
            pl.BlockSpec((N,), lambda i: (0,)),
        ],
        out_specs=pl.BlockSpec((ROW_BLK, N), lambda i: (i, 0)),
        out_shape=jax.ShapeDtypeStruct((N, N), jnp.float32),
    )(h, fc1_w, fc1_b, fc2_w.astype(jnp.bfloat16), fc2_b)


def kernel(x, edge_index, edge_attr, W1, root1, b1, W2, root2, b2, W3, root3,
           b3, W4, root4, b4, W5, root5, b5, W6, root6, b6, fc1_w, fc1_b,
           fc2_w, fc2_b):
    src = edge_index[0]
    dst = edge_index[1]
    src3 = src.reshape(NCH, 1, C)
    a0 = edge_attr[:, 0].reshape(NCH, 1, C)
    a1 = edge_attr[:, 1].reshape(NCH, 1, C)
    a2 = edge_attr[:, 2].reshape(NCH, 1, C)

    b8, idn, iw = _prep(src3, a0, a1, a2)
    b8f = b8.reshape(8 * E)
    idnf = idn.reshape(8 * E)
    iwf = iw.reshape(8 * E)

    zer16 = jnp.zeros((NP, 16), jnp.float32)
    zer32 = jnp.zeros((NP, 32), jnp.float32)
    zer64 = jnp.zeros((NP, 64), jnp.float32)
    ones16 = jnp.ones((C, 16), jnp.float32)

    degp = _deg_kernel(src, ones16, zer16)[:, :N, :]

    agg = _sc_layer1(W1[:, 0, :], iwf, b8f, dst, zer32)[:, :N, :]
    h = _act(x, agg, root1, b1, degp)

    for (W, root, b) in ((W2, root2, b2), (W3, root3, b3), (W4, root4, b4),
                         (W5, root5, b5), (W6, root6, b6)):
        din = W.shape[1]
        Wr = W.transpose(1, 0, 2).reshape(din, 125 * 64)
        T = _mm(h, Wr).reshape(N * 125, 64)
        agg = _sc_layer64(T, idnf, b8f, dst, zer64)[:, :N, :]
        h = _act(h, agg, root, b, degp)

    return _head(h, fc1_w, fc1_b, fc2_w, fc2_b)


# speedup vs baseline: 1.2212x; 1.0086x over previous
import functools

import jax
import jax.numpy as jnp
from jax import lax
from jax.experimental import pallas as pl
from jax.experimental.pallas import tpu as pltpu
from jax.experimental.pallas import tpu_sc as plsc

N = 10000
E = 160000
K = 5
C = 128
NW = 32
NCH = E // C
NP = 10240
ROWS = NP // 16

ROW_BLK = 400


def _prep_body(src_ref, a0_ref, a1_ref, a2_ref, b_ref, idn_ref, iw_ref):
    src = src_ref[:, 0, :]
    fr = []
    fi = []
    for a_ref in (a0_ref, a1_ref, a2_ref):
        v = a_ref[:, 0, :] * (K - 1.0)
        fl = jnp.floor(v)
        fr.append(v - fl)
        fi.append(fl.astype(jnp.int32))
    for s in range(8):
        b = None
        wi = None
        stride = 1
        for d in range(3):
            bit = (s >> d) & 1
            bd = fr[d] if bit else 1.0 - fr[d]
            wd = jnp.clip(fi[d] + bit, 0, K - 1) * stride
            b = bd if b is None else b * bd
            wi = wd if wi is None else wi + wd
            stride *= K
        b_ref[:, s, :] = b
        iw_ref[:, s, :] = wi
        idn_ref[:, s, :] = jnp.minimum(src, N - 1) * 125 + wi


def _prep(src3, a0, a1, a2):
    blk = 10
    return pl.pallas_call(
        _prep_body,
        grid=(NCH // blk,),
        in_specs=[pl.BlockSpec((blk, 1, C), lambda i: (i, 0, 0))] * 4,
        out_specs=[pl.BlockSpec((blk, 8, C), lambda i: (i, 0, 0))] * 3,
        out_shape=[
            jax.ShapeDtypeStruct((NCH, 8, C), jnp.float32),
            jax.ShapeDtypeStruct((NCH, 8, C), jnp.int32),
            jax.ShapeDtypeStruct((NCH, 8, C), jnp.int32),
        ],
    )(src3, a0, a1, a2)


_MESH = plsc.VectorSubcoreMesh(core_axis_name="c", subcore_axis_name="s")


def _worker_id():
    return lax.axis_index("s") * 2 + lax.axis_index("c")


def _nchunks(w):
    return jnp.where(w < NCH - (NCH // NW) * NW, NCH // NW + 1, NCH // NW)


@functools.partial(
    pl.kernel,
    out_type=jax.ShapeDtypeStruct((2, NP, 16), jnp.float32),
    mesh=_MESH,
    compiler_params=pltpu.CompilerParams(use_tc_tiling_on_sc=False),
    scratch_types=[
        pltpu.VMEM((C,), jnp.int32),
        pltpu.VMEM((C, 16), jnp.float32),
        pltpu.VMEM_SHARED((NP, 16), jnp.float32),
    ],
)
def _deg_kernel(srcf_hbm, ones_hbm, zer_hbm, degp_hbm, src_v, ones_v, dacc):
    cid = lax.axis_index("c")
    sid = lax.axis_index("s")
    w = _worker_id()
    pltpu.sync_copy(ones_hbm, ones_v)
    pltpu.sync_copy(zer_hbm.at[pl.ds(sid * ROWS, ROWS)],
                    dacc.at[pl.ds(sid * ROWS, ROWS)])
    plsc.subcore_barrier()

    def chunk(i, carry):
        ch = w + i * NW
        pltpu.sync_copy(srcf_hbm.at[pl.ds(ch * C, C)], src_v)
        pltpu.sync_copy(ones_v, dacc.at[src_v], add=True)
        return carry

    lax.fori_loop(0, _nchunks(w), chunk, 0)
    plsc.subcore_barrier()
    pltpu.sync_copy(dacc.at[pl.ds(sid * ROWS, ROWS)],
                    degp_hbm.at[cid, pl.ds(sid * ROWS, ROWS)])


def _make_sc_layer(D):

    @functools.partial(
        pl.kernel,
        out_type=jax.ShapeDtypeStruct((2, NP, D), jnp.float32),
        mesh=_MESH,
        compiler_params=pltpu.CompilerParams(use_tc_tiling_on_sc=False),
        scratch_types=[
            pltpu.VMEM((8 * C,), jnp.int32),
            pltpu.VMEM((8 * C,), jnp.float32),
            pltpu.VMEM((C,), jnp.int32),
            [pltpu.VMEM((C, D), jnp.float32) for _ in range(8)],
            pltpu.VMEM((C, D), jnp.float32),
            pltpu.VMEM_SHARED((NP, D), jnp.float32),
            pltpu.SemaphoreType.DMA,
            pltpu.SemaphoreType.DMA,
        ],
    )
    def sc_layer(table_hbm, idx_hbm, b_hbm, dst_hbm, zer_hbm, agg_hbm,
                 idx_v, b_v, dst_v, gb, msg_v, acc, sem, msem):
        cid = lax.axis_index("c")
        sid = lax.axis_index("s")
        w = _worker_id()
        pltpu.sync_copy(zer_hbm.at[pl.ds(sid * ROWS, ROWS)],
                        acc.at[pl.ds(sid * ROWS, ROWS)])
        plsc.subcore_barrier()

        def chunk(i, carry):
            ch = w + i * NW
            mps = [
                pltpu.async_copy(
                    idx_hbm.at[pl.ds(ch * (8 * C), 8 * C)], idx_v, msem),
                pltpu.async_copy(
                    b_hbm.at[pl.ds(ch * (8 * C), 8 * C)], b_v, msem),
                pltpu.async_copy(dst_hbm.at[pl.ds(ch * C, C)], dst_v, msem),
            ]
            for mp in mps:
                mp.wait()
            cps = [
                pltpu.async_copy(
                    table_hbm.at[idx_v.at[pl.ds(s * C, C)]], gb[s], sem)
                for s in range(8)
            ]
            for cp in cps:
                cp.wait()

            def egroup(eg, ecarry):
                bvs = [b_v[pl.ds(s * C + eg * 16, 16)] for s in range(8)]
                for j in range(16):
                    e = eg * 16 + j
                    bbs = [lax.broadcast(bvs[s][j], (16,)) for s in range(8)]
                    for g in range(D // 16):
                        sl = pl.ds(g * 16, 16)
                        accv = bbs[0] * gb[0][e, sl]
                        for s in range(1, 8):
                            accv = accv + bbs[s] * gb[s][e, sl]
                        msg_v[e, sl] = accv
                return ecarry

            lax.fori_loop(0, C // 16, egroup, 0)
            pltpu.sync_copy(msg_v, acc.at[dst_v], add=True)
            return carry

        lax.fori_loop(0, _nchunks(w), chunk, 0)
        plsc.subcore_barrier()
        pltpu.sync_copy(acc.at[pl.ds(sid * ROWS, ROWS)],
                        agg_hbm.at[cid, pl.ds(sid * ROWS, ROWS)])

    return sc_layer


_sc_layer64 = _make_sc_layer(64)


@functools.partial(
    pl.kernel,
    out_type=jax.ShapeDtypeStruct((2, NP, 32), jnp.float32),
    mesh=_MESH,
    compiler_params=pltpu.CompilerParams(use_tc_tiling_on_sc=False),
    scratch_types=[
        pltpu.VMEM((125, 32), jnp.float32),
        pltpu.VMEM((8 * C,), jnp.int32),
        pltpu.VMEM((8 * C,), jnp.float32),
        pltpu.VMEM((C,), jnp.int32),
        pltpu.VMEM((C, 32), jnp.float32),
        pltpu.VMEM_SHARED((NP, 32), jnp.float32),
        pltpu.SemaphoreType.DMA,
    ],
)
def _sc_layer1(table_hbm, idx_hbm, b_hbm, dst_hbm, zer_hbm, agg_hbm,
               tb, idx_v, b_v, dst_v, msg_v, acc, msem):
    cid = lax.axis_index("c")
    sid = lax.axis_index("s")
    w = _worker_id()
    pltpu.sync_copy(table_hbm, tb)
    pltpu.sync_copy(zer_hbm.at[pl.ds(sid * ROWS, ROWS)],
                    acc.at[pl.ds(sid * ROWS, ROWS)])
    plsc.subcore_barrier()

    def chunk(i, carry):
        ch = w + i * NW
        mps = [
            pltpu.async_copy(
                idx_hbm.at[pl.ds(ch * (8 * C), 8 * C)], idx_v, msem),
            pltpu.async_copy(
                b_hbm.at[pl.ds(ch * (8 * C), 8 * C)], b_v, msem),
            pltpu.async_copy(dst_hbm.at[pl.ds(ch * C, C)], dst_v, msem),
        ]
        for mp in mps:
            mp.wait()

        def egroup(eg, ecarry):
            bvs = [b_v[pl.ds(s * C + eg * 16, 16)] for s in range(8)]
            wvs = [idx_v[pl.ds(s * C + eg * 16, 16)] for s in range(8)]
            for j in range(16):
                e = eg * 16 + j
                bbs = [lax.broadcast(bvs[s][j], (16,)) for s in range(8)]
                wis = [wvs[s][j] for s in range(8)]
                for g in range(2):
                    sl = pl.ds(g * 16, 16)
                    accv = bbs[0] * tb[wis[0], sl]
                    for s in range(1, 8):
                        accv = accv + bbs[s] * tb[wis[s], sl]
                    msg_v[e, sl] = accv
            return ecarry

        lax.fori_loop(0, C // 16, egroup, 0)
        pltpu.sync_copy(msg_v, acc.at[dst_v], add=True)
        return carry

    lax.fori_loop(0, _nchunks(w), chunk, 0)
    plsc.subcore_barrier()
    pltpu.sync_copy(acc.at[pl.ds(sid * ROWS, ROWS)],
                    agg_hbm.at[cid, pl.ds(sid * ROWS, ROWS)])


def _mm_body(h_ref, w_ref, out_ref):
    out_ref[...] = jnp.dot(h_ref[...], w_ref[...],
                           preferred_element_type=jnp.float32)


def _mm(h, Wr):
    din = h.shape[1]
    cols = Wr.shape[1]
    return pl.pallas_call(
        _mm_body,
        grid=(N // ROW_BLK,),
        in_specs=[
            pl.BlockSpec((ROW_BLK, din), lambda i: (i, 0)),
            pl.BlockSpec((din, cols), lambda i: (0, 0)),
        ],
        out_specs=pl.BlockSpec((ROW_BLK, cols), lambda i: (i, 0)),
        out_shape=jax.ShapeDtypeStruct((N, cols), jnp.float32),
    )(h, Wr)


def _act_body(h_ref, agg_ref, root_ref, bias_ref, degp_ref, out_ref):
    deg = 1.0 + degp_ref[0, :, 0:1] + degp_ref[1, :, 0:1]
    a = agg_ref[0] + agg_ref[1] + bias_ref[...]
    a = a + jnp.dot(h_ref[...], root_ref[...],
                    preferred_element_type=jnp.float32)
    a = a / deg
    out_ref[...] = jnp.where(a > 0, a, jnp.exp(a) - 1.0)


def _act(h, agg, root, bias, degp):
    din = h.shape[1]
    d = agg.shape[2]
    return pl.pallas_call(
        _act_body,
        in_specs=[
            pl.BlockSpec((N, din), lambda: (0, 0)),
            pl.BlockSpec((2, N, d), lambda: (0, 0, 0)),
            pl.BlockSpec((din, d), lambda: (0, 0)),
            pl.BlockSpec((d,), lambda: (0,)),
            pl.BlockSpec((2, N, 16), lambda: (0, 0, 0)),
        ],
        out_specs=pl.BlockSpec((N, d), lambda: (0, 0)),
        out_shape=jax.ShapeDtypeStruct((N, d), jnp.float32),
        grid=(),
    )(h, agg, root, bias, degp)


def _head_body(h_ref, fc1w_ref, fc1b_ref, fc2w_ref, fc2b_ref, out_ref):
    h = h_ref[...]
    a = jnp.dot(h, fc1w_ref[...], preferred_element_type=jnp.float32)
    a = a + fc1b_ref[...]
    a = jnp.where(a > 0, a, jnp.exp(a) - 1.0)
    logits = jnp.dot(a.astype(jnp.bfloat16), fc2w_ref[...],
                      preferred_element_type=jnp.float32)
    logits = logits + fc2b_ref[...]
    m = jnp.max(logits, axis=1, keepdims=True)
    lse = jnp.log(jnp.sum(jnp.exp(logits - m), axis=1, keepdims=True))
    out_ref[...] = logits - m - lse


def _head(h, fc1_w, fc1_b, fc2_w, fc2_b):
    return pl.pallas_call(
        _head_body,
        grid=(N // ROW_BLK,),
        in_specs=[
            pl.BlockSpec((ROW_BLK, 64), lambda i: (i, 0)),
            pl.BlockSpec((64, 256), lambda i: (0, 0)),
            pl.BlockSpec((256,), lambda i: (0,)),
            pl.BlockSpec((256, N), lambda i: (0, 0)),
            pl.BlockSpec((N,), lambda i: (0,)),
        ],
        out_specs=pl.BlockSpec((ROW_BLK, N), lambda i: (i, 0)),
        out_shape=jax.ShapeDtypeStruct((N, N), jnp.float32),
    )(h, fc1_w, fc1_b, fc2_w.astype(jnp.bfloat16), fc2_b)


def kernel(x, edge_index, edge_attr, W1, root1, b1, W2, root2, b2, W3, root3,
           b3, W4, root4, b4, W5, root5, b5, W6, root6, b6, fc1_w, fc1_b,
           fc2_w, fc2_b):
    src = edge_index[0]
    dst = edge_index[1]
    src3 = src.reshape(NCH, 1, C)
    a0 = edge_attr[:, 0].reshape(NCH, 1, C)
    a1 = edge_attr[:, 1].reshape(NCH, 1, C)
    a2 = edge_attr[:, 2].reshape(NCH, 1, C)

    b8, idn, iw = _prep(src3, a0, a1, a2)
    b8f = b8.reshape(8 * E)
    idnf = idn.reshape(8 * E)
    iwf = iw.reshape(8 * E)

    zer16 = jnp.zeros((NP, 16), jnp.float32)
    zer32 = jnp.zeros((NP, 32), jnp.float32)
    zer64 = jnp.zeros((NP, 64), jnp.float32)
    ones16 = jnp.ones((C, 16), jnp.float32)

    degp = _deg_kernel(src, ones16, zer16)[:, :N, :]

    agg = _sc_layer1(W1[:, 0, :], iwf, b8f, dst, zer32)[:, :N, :]
    h = _act(x, agg, root1, b1, degp)

    for (W, root, b) in ((W2, root2, b2), (W3, root3, b3), (W4, root4, b4),
                         (W5, root5, b5), (W6, root6, b6)):
        din = W.shape[1]
        Wr = W.transpose(1, 0, 2).reshape(din, 125 * 64)
        T = _mm(h, Wr).reshape(N * 125, 64)
        agg = _sc_layer64(T, idnf, b8f, dst, zer64)[:, :N, :]
        h = _act(h, agg, root, b, degp)

    return _head(h, fc1_w, fc1_b, fc2_w, fc2_b)
